# Initial kernel scaffold; baseline (speedup 1.0000x reference)
import jax
import jax.numpy as jnp
from jax.experimental import pallas as pl


def kernel(token_has_int, token_signed_norm, token_log_norm, token_is_zero, token_is_one, token_is_pow2, var_family_onehot, var_outer_norm, var_inner_norm, var_has_outer, var_has_inner, ln_gamma, ln_beta, w1, b1, w2, b2, scale, token_ids, var_ids, var_family_id, var_group_id):
    raise NotImplementedError("write your pallas kernel here")


# speedup vs baseline: 25.3111x; 25.3111x over previous
import functools

import numpy as np
import jax
import jax.numpy as jnp
from jax import lax
from jax.experimental import pallas as pl
from jax.experimental.pallas import tpu as pltpu
from jax.experimental.pallas import tpu_sc as plsc

_B = 1024
_S = 200
_N = _B * _S
_BB = 8
_HID = 64
_DM = 128
_K = 128

_POS = np.array([0, 1, 2, 3, 5,
                 16, 17, 18, 19, 20, 21, 22, 23,
                 8, 9, 10, 11,
                 24,
                 25, 26, 27, 28, 29, 30, 31, 32, 33, 34,
                 4], dtype=np.int32)
_F = 35
_UPPER = np.triu(np.ones((_S, _S), np.float32), 1)


def _sc_gather(tok_tab, var_tab, tok_idx, var_idx):
    info = plsc.get_sparse_core_info()
    nc, ns = info.num_cores, info.num_subcores
    nw = nc * ns
    per_w = _N // nw

    @functools.partial(
        pl.kernel,
        mesh=plsc.VectorSubcoreMesh(core_axis_name="c", subcore_axis_name="s"),
        compiler_params=pltpu.CompilerParams(use_tc_tiling_on_sc=False),
        out_type=(jax.ShapeDtypeStruct((_N, 8), jnp.float32),
                  jax.ShapeDtypeStruct((_N, 8), jnp.float32)),
        scratch_types=[pltpu.VMEM((per_w,), jnp.int32),
                       pltpu.VMEM((per_w, 8), jnp.float32),
                       pltpu.VMEM((per_w,), jnp.int32),
                       pltpu.VMEM((per_w, 8), jnp.float32),
                       pltpu.SemaphoreType.DMA,
                       pltpu.SemaphoreType.DMA],
    )
    def gk(tok_tab_h, var_tab_h, tok_idx_h, var_idx_h, tok_out, var_out,
           ti_v, tr_v, vi_v, vr_v, s1, s2):
        wid = lax.axis_index("s") * nc + lax.axis_index("c")
        base = wid * per_w
        pltpu.sync_copy(tok_idx_h.at[pl.ds(base, per_w)], ti_v)
        pltpu.sync_copy(var_idx_h.at[pl.ds(base, per_w)], vi_v)

        def body(j, carry):
            o = j * _K
            c1 = pltpu.async_copy(tok_tab_h.at[ti_v.at[pl.ds(o, _K)]],
                                  tr_v.at[pl.ds(o, _K)], s1)
            c2 = pltpu.async_copy(var_tab_h.at[vi_v.at[pl.ds(o, _K)]],
                                  vr_v.at[pl.ds(o, _K)], s2)
            c1.wait()
            c2.wait()
            return carry

        lax.fori_loop(0, per_w // _K, body, 0)
        pltpu.sync_copy(tr_v, tok_out.at[pl.ds(base, per_w)])
        pltpu.sync_copy(vr_v, var_out.at[pl.ds(base, per_w)])

    return gk(tok_tab, var_tab, tok_idx, var_idx)


def _tc_body(tok_ref, var_ref, tid_ref, u_ref, w1_ref, b1_ref, w2_ref, b2_ref,
             g_ref, be_ref, mk_ref, sc_ref, out_ref):
    f32 = jnp.float32
    inv = np.float32(1.0 / (_S - 1))
    U = u_ref[...]
    tid = tid_ref[...]
    tchunks = [tok_ref[b] for b in range(_BB)]
    vchunks = [var_ref[b] for b in range(_BB)]
    Ts = [jnp.concatenate([tchunks[b], vchunks[b]], axis=1).T
          for b in range(_BB)]

    def rows(i):
        return jnp.concatenate([Ts[b][i:i + 1] for b in range(_BB)], axis=0)

    hi, sg, lg = rows(0), rows(1), rows(2)
    famf, grpf = rows(12), rows(13)
    valid = (hi > 0.0) & (tid != 0) & (tid != 1) & (tid != 2)
    vf = valid.astype(f32)
    lv = lg * vf
    famoh = [(famf == np.float32(f)).astype(f32) for f in range(8)]
    grpoh = [(grpf == np.float32(g)).astype(f32) for g in range(16)]
    gmask = (grpf > 0.0).astype(f32)
    vg = vf * gmask
    lvg = lv * gmask
    X = jnp.concatenate(
        [vf, lv]
        + [famoh[f] * vf for f in range(8)]
        + [famoh[f] * lv for f in range(8)]
        + [grpoh[g] * vg for g in range(16)]
        + [grpoh[g] * lvg for g in range(16)], axis=0)
    prev = jnp.dot(X, U, preferred_element_type=f32)
    pc = prev[0:_BB]
    pls = prev[_BB:2 * _BB]
    psfc = sum(prev[(2 + f) * _BB:(3 + f) * _BB] * famoh[f] for f in range(8))
    psfl = sum(prev[(10 + f) * _BB:(11 + f) * _BB] * famoh[f] for f in range(8))
    psgc = sum(prev[(18 + g) * _BB:(19 + g) * _BB] * grpoh[g] for g in range(16))
    psgl = sum(prev[(34 + g) * _BB:(35 + g) * _BB] * grpoh[g] for g in range(16))
    rden = 1.0 / jnp.maximum(pc, 1.0)
    z1 = jnp.zeros((_BB, 1), f32)
    dyn10 = [pc * inv, pls * inv, psfc * inv, psfl * inv, psfc * rden,
             psgc * inv, psgl * inv, psgc * rden,
             jnp.concatenate([z1, lg[:, :_S - 1]], axis=1),
             jnp.concatenate([z1, sg[:, :_S - 1]], axis=1)]
    posb = lax.broadcasted_iota(jnp.int32, (_S, 1), 0).astype(f32) * inv
    ioq = lax.broadcasted_iota(jnp.int32, (_S, 8), 1).astype(f32)
    feats_list = []
    for b in range(_BB):
        dynb = jnp.concatenate([p[b:b + 1] for p in dyn10], axis=0)
        famohb = (vchunks[b][:, 4:5] == ioq).astype(f32)
        fb = jnp.concatenate([tchunks[b], vchunks[b], famohb, posb, dynb.T],
                             axis=1)
        feats_list.append(fb)
    feats = jnp.concatenate(feats_list, axis=0)
    m = mk_ref[...]
    fm = feats * m
    mu = jnp.sum(fm, axis=1, keepdims=True) * np.float32(1.0 / 29.0)
    ex2 = jnp.sum(fm * fm, axis=1, keepdims=True) * np.float32(1.0 / 29.0)
    varr = ex2 - mu * mu
    x = (feats - mu) * lax.rsqrt(varr + 1e-5) * g_ref[...] + be_ref[...]
    h = jnp.dot(x, w1_ref[...], preferred_element_type=f32) + b1_ref[...]
    act = 0.5 * h * (1.0 + lax.erf(h * np.float32(1.0 / np.sqrt(2.0))))
    o = jnp.dot(act, w2_ref[...], preferred_element_type=f32) + b2_ref[...]
    o = o * sc_ref[0, 0]
    for b in range(_BB):
        out_ref[b] = o[b * _S:(b + 1) * _S, :]


def _tc_forward(tok3, var3, token_ids, u, w1p, b1r, w2, b2r, gp, bp, mk, scr):
    cb = lambda shape: pl.BlockSpec(shape, lambda i: (0,) * len(shape))
    return pl.pallas_call(
        _tc_body,
        grid=(_B // _BB,),
        in_specs=[
            pl.BlockSpec((_BB, _S, 8), lambda i: (i, 0, 0)),
            pl.BlockSpec((_BB, _S, 8), lambda i: (i, 0, 0)),
            pl.BlockSpec((_BB, _S), lambda i: (i, 0)),
            cb((_S, _S)),
            cb((_F, _HID)),
            cb((1, _HID)),
            cb((_HID, _DM)),
            cb((1, _DM)),
            cb((1, _F)),
            cb((1, _F)),
            cb((1, _F)),
            cb((1, 1)),
        ],
        out_specs=pl.BlockSpec((_BB, _S, _DM), lambda i: (i, 0, 0)),
        out_shape=jax.ShapeDtypeStruct((_B, _S, _DM), jnp.float32),
    )(tok3, var3, token_ids, u, w1p, b1r, w2, b2r, gp, bp, mk, scr)


def kernel(token_has_int, token_signed_norm, token_log_norm, token_is_zero,
           token_is_one, token_is_pow2, var_family_onehot, var_outer_norm,
           var_inner_norm, var_has_outer, var_has_inner, ln_gamma, ln_beta,
           w1, b1, w2, b2, scale, token_ids, var_ids, var_family_id,
           var_group_id):
    f32 = jnp.float32
    zv = jnp.zeros_like(token_has_int)
    tok_tab = jnp.stack([token_has_int, token_signed_norm, token_log_norm,
                         token_is_zero, token_is_one, token_is_pow2, zv, zv],
                        axis=1)
    zn = jnp.zeros_like(var_outer_norm)
    var_tab = jnp.stack([var_outer_norm, var_inner_norm, var_has_outer,
                         var_has_inner, var_family_id.astype(f32),
                         var_group_id.astype(f32), zn, zn], axis=1)
    tok_g, var_g = _sc_gather(tok_tab, var_tab,
                              token_ids.reshape(-1), var_ids.reshape(-1))
    tok3 = tok_g.reshape(_B, _S, 8)
    var3 = var_g.reshape(_B, _S, 8)
    pos = jnp.asarray(_POS)
    w1p = jnp.zeros((_F, _HID), f32).at[pos].set(w1)
    gp = jnp.zeros((_F,), f32).at[pos].set(ln_gamma).reshape(1, _F)
    bp = jnp.zeros((_F,), f32).at[pos].set(ln_beta).reshape(1, _F)
    mk = jnp.zeros((_F,), f32).at[pos].set(1.0).reshape(1, _F)
    return _tc_forward(tok3, var3, token_ids, jnp.asarray(_UPPER), w1p,
                       b1.reshape(1, _HID), w2, b2.reshape(1, _DM), gp, bp, mk,
                       jnp.asarray(scale, f32).reshape(1, 1))


# speedup vs baseline: 28.6101x; 1.1303x over previous
import functools

import numpy as np
import jax
import jax.numpy as jnp
from jax import lax
from jax.experimental import pallas as pl
from jax.experimental.pallas import tpu as pltpu
from jax.experimental.pallas import tpu_sc as plsc

_B = 1024
_S = 200
_N = _B * _S
_BB = 8
_HID = 64
_DM = 128
_K = 128

_MAP_TOK = np.array([0, 1, 2, 3, 28, 4, -1, -1], dtype=np.int32)
_MAP_VAR = np.array([13, 14, 15, 16, -1, -1, -1, -1], dtype=np.int32)
_MAP_FAM = np.arange(5, 13, dtype=np.int32)
_MAP_DYN = np.arange(17, 28, dtype=np.int32)
_UPPER = np.triu(np.ones((_S, _S), np.float32), 1)


def _sc_gather(tok_tab, var_tab, tok_idx, var_idx):
    info = plsc.get_sparse_core_info()
    nc, ns = info.num_cores, info.num_subcores
    nw = nc * ns
    per_w = _N // nw

    @functools.partial(
        pl.kernel,
        mesh=plsc.VectorSubcoreMesh(core_axis_name="c", subcore_axis_name="s"),
        compiler_params=pltpu.CompilerParams(use_tc_tiling_on_sc=False),
        out_type=(jax.ShapeDtypeStruct((_N, 8), jnp.float32),
                  jax.ShapeDtypeStruct((_N, 8), jnp.float32)),
        scratch_types=[pltpu.VMEM((per_w,), jnp.int32),
                       pltpu.VMEM((per_w, 8), jnp.float32),
                       pltpu.VMEM((per_w,), jnp.int32),
                       pltpu.VMEM((per_w, 8), jnp.float32),
                       pltpu.SemaphoreType.DMA,
                       pltpu.SemaphoreType.DMA],
    )
    def gk(tok_tab_h, var_tab_h, tok_idx_h, var_idx_h, tok_out, var_out,
           ti_v, tr_v, vi_v, vr_v, s1, s2):
        wid = lax.axis_index("s") * nc + lax.axis_index("c")
        base = wid * per_w
        pltpu.sync_copy(tok_idx_h.at[pl.ds(base, per_w)], ti_v)
        pltpu.sync_copy(var_idx_h.at[pl.ds(base, per_w)], vi_v)

        def body(j, carry):
            o = j * _K
            c1 = pltpu.async_copy(tok_tab_h.at[ti_v.at[pl.ds(o, _K)]],
                                  tr_v.at[pl.ds(o, _K)], s1)
            c2 = pltpu.async_copy(var_tab_h.at[vi_v.at[pl.ds(o, _K)]],
                                  vr_v.at[pl.ds(o, _K)], s2)
            c1.wait()
            c2.wait()
            return carry

        lax.fori_loop(0, per_w // _K, body, 0)
        pltpu.sync_copy(tr_v, tok_out.at[pl.ds(base, per_w)])
        pltpu.sync_copy(vr_v, var_out.at[pl.ds(base, per_w)])

    return gk(tok_tab, var_tab, tok_idx, var_idx)


def _tc_body(tok_ref, var_ref, tid_ref, u_ref, wa_ref, wb_ref, wc_ref, wd_ref,
             cw_ref, bw_ref, w2_ref, b2_ref, out_ref):
    f32 = jnp.float32
    inv = np.float32(1.0 / (_S - 1))
    U = u_ref[...]
    tid = tid_ref[...]
    tchunks = [tok_ref[b] for b in range(_BB)]
    vchunks = [var_ref[b] for b in range(_BB)]
    Ts = [jnp.concatenate([tchunks[b], vchunks[b]], axis=1).T
          for b in range(_BB)]

    def rows(i):
        return jnp.concatenate([Ts[b][i:i + 1] for b in range(_BB)], axis=0)

    hi, sg, lg = rows(0), rows(1), rows(2)
    famf, grpf = rows(12), rows(13)
    valid = (hi > 0.0) & (tid != 0) & (tid != 1) & (tid != 2)
    vf = valid.astype(f32)
    lv = lg * vf
    famoh = [(famf == np.float32(f)).astype(f32) for f in range(8)]
    grpoh = [(grpf == np.float32(g)).astype(f32) for g in range(16)]
    gmask = (grpf > 0.0).astype(f32)
    vg = vf * gmask
    lvg = lv * gmask
    X = jnp.concatenate(
        [vf, lv]
        + [famoh[f] * vf for f in range(8)]
        + [famoh[f] * lv for f in range(8)]
        + [grpoh[g] * vg for g in range(16)]
        + [grpoh[g] * lvg for g in range(16)], axis=0)
    prev = jnp.dot(X, U, preferred_element_type=f32)
    pc = prev[0:_BB]
    pls = prev[_BB:2 * _BB]
    psfc = sum(prev[(2 + f) * _BB:(3 + f) * _BB] * famoh[f] for f in range(8))
    psfl = sum(prev[(10 + f) * _BB:(11 + f) * _BB] * famoh[f] for f in range(8))
    psgc = sum(prev[(18 + g) * _BB:(19 + g) * _BB] * grpoh[g] for g in range(16))
    psgl = sum(prev[(34 + g) * _BB:(35 + g) * _BB] * grpoh[g] for g in range(16))
    rden = 1.0 / jnp.maximum(pc, 1.0)
    z1 = jnp.zeros((_BB, 1), f32)
    pos = lax.broadcasted_iota(jnp.int32, (_BB, _S), 1).astype(f32) * inv
    dyn11 = [pos, pc * inv, pls * inv, psfc * inv, psfl * inv, psfc * rden,
             psgc * inv, psgl * inv, psgc * rden,
             jnp.concatenate([z1, lg[:, :_S - 1]], axis=1),
             jnp.concatenate([z1, sg[:, :_S - 1]], axis=1)]
    dynT = jnp.concatenate(
        [jnp.concatenate([p[b:b + 1] for p in dyn11], axis=0).T
         for b in range(_BB)], axis=0)
    tok = tok_ref[...].reshape(_BB * _S, 8)
    var = var_ref[...].reshape(_BB * _S, 8)
    ioq = lax.broadcasted_iota(jnp.int32, (1, 8), 1).astype(f32)
    fam_rm = (var[:, 4:5] == ioq).astype(f32)
    wa, wb, wc, wd = wa_ref[...], wb_ref[...], wc_ref[...], wd_ref[...]
    hmu = (jnp.dot(tok, wa, preferred_element_type=f32)
           + jnp.dot(var, wb, preferred_element_type=f32)
           + jnp.dot(fam_rm, wc, preferred_element_type=f32)
           + jnp.dot(dynT, wd, preferred_element_type=f32))
    mu = hmu[:, _HID:_HID + 1]
    ex2 = (jnp.dot(tok * tok, wa[:, _HID:], preferred_element_type=f32)
           + jnp.dot(var * var, wb[:, _HID:], preferred_element_type=f32)
           + jnp.dot(fam_rm, wc[:, _HID:], preferred_element_type=f32)
           + jnp.dot(dynT * dynT, wd[:, _HID:], preferred_element_type=f32))
    r = lax.rsqrt(ex2 - mu * mu + 1e-5)
    h = (hmu[:, :_HID] - mu * cw_ref[...]) * r + bw_ref[...]
    act = 0.5 * h * (1.0 + lax.erf(h * np.float32(1.0 / np.sqrt(2.0))))
    o = jnp.dot(act, w2_ref[...], preferred_element_type=f32) + b2_ref[...]
    for b in range(_BB):
        out_ref[b] = o[b * _S:(b + 1) * _S, :]


def _tc_forward(tok3, var3, token_ids, u, wa, wb, wc, wd, cw, bw, w2s, b2s):
    cb = lambda shape: pl.BlockSpec(shape, lambda i: (0,) * len(shape))
    return pl.pallas_call(
        _tc_body,
        grid=(_B // _BB,),
        in_specs=[
            pl.BlockSpec((_BB, _S, 8), lambda i: (i, 0, 0)),
            pl.BlockSpec((_BB, _S, 8), lambda i: (i, 0, 0)),
            pl.BlockSpec((_BB, _S), lambda i: (i, 0)),
            cb((_S, _S)),
            cb((8, _HID + 1)),
            cb((8, _HID + 1)),
            cb((8, _HID + 1)),
            cb((11, _HID + 1)),
            cb((1, _HID)),
            cb((1, _HID)),
            cb((_HID, _DM)),
            cb((1, _DM)),
        ],
        out_specs=pl.BlockSpec((_BB, _S, _DM), lambda i: (i, 0, 0)),
        out_shape=jax.ShapeDtypeStruct((_B, _S, _DM), jnp.float32),
    )(tok3, var3, token_ids, u, wa, wb, wc, wd, cw, bw, w2s, b2s)


def kernel(token_has_int, token_signed_norm, token_log_norm, token_is_zero,
           token_is_one, token_is_pow2, var_family_onehot, var_outer_norm,
           var_inner_norm, var_has_outer, var_has_inner, ln_gamma, ln_beta,
           w1, b1, w2, b2, scale, token_ids, var_ids, var_family_id,
           var_group_id):
    f32 = jnp.float32
    zv = jnp.zeros_like(token_has_int)
    tok_tab = jnp.stack([token_has_int, token_signed_norm, token_log_norm,
                         token_is_zero, token_is_one, token_is_pow2, zv, zv],
                        axis=1)
    zn = jnp.zeros_like(var_outer_norm)
    var_tab = jnp.stack([var_outer_norm, var_inner_norm, var_has_outer,
                         var_has_inner, var_family_id.astype(f32),
                         var_group_id.astype(f32), zn, zn], axis=1)
    tok_g, var_g = _sc_gather(tok_tab, var_tab,
                              token_ids.reshape(-1), var_ids.reshape(-1))
    tok3 = tok_g.reshape(_B, _S, 8)
    var3 = var_g.reshape(_B, _S, 8)
    wts = _prep_weights(ln_gamma, ln_beta, w1, b1, w2, b2, scale)
    return _tc_forward(tok3, var3, token_ids, jnp.asarray(_UPPER), *wts)


def _prep_weights(ln_gamma, ln_beta, w1, b1, w2, b2, scale):
    f32 = jnp.float32
    w1g = w1 * ln_gamma[:, None]

    def piece(mapping):
        mp = jnp.asarray(mapping)
        use = (mp >= 0).astype(f32)[:, None]
        sel = jnp.take(w1g, jnp.maximum(mp, 0), axis=0) * use
        return jnp.concatenate([sel, use * np.float32(1.0 / 29.0)], axis=1)

    cw = (ln_gamma @ w1).reshape(1, _HID)
    bw = (ln_beta @ w1 + b1).reshape(1, _HID)
    return (piece(_MAP_TOK), piece(_MAP_VAR), piece(_MAP_FAM), piece(_MAP_DYN),
            cw, bw, w2 * scale, (b2 * scale).reshape(1, _DM))


# speedup vs baseline: 31.3484x; 1.0957x over previous
import functools

import numpy as np
import jax
import jax.numpy as jnp
from jax import lax
from jax.experimental import pallas as pl
from jax.experimental.pallas import tpu as pltpu
from jax.experimental.pallas import tpu_sc as plsc

_B = 1024
_S = 200
_N = _B * _S
_BB = 8
_HID = 64
_DM = 128
_K = 128

_MAP_TOK = np.array([0, 1, 2, 3, 28, 4, -1, -1], dtype=np.int32)
_MAP_VAR = np.array([13, 14, 15, 16, -1, -1, -1, -1], dtype=np.int32)
_MAP_FAM = np.arange(5, 13, dtype=np.int32)
_MAP_DYN = np.arange(18, 28, dtype=np.int32)
_UPPER = np.triu(np.ones((_S, _S), np.float32), 1)
_POSCOL = ((np.arange(_BB * _S) % _S).astype(np.float32) / (_S - 1))[:, None]


def _sc_gather(tok_tab, var_tab, tok_idx, var_idx):
    info = plsc.get_sparse_core_info()
    nc, ns = info.num_cores, info.num_subcores
    nw = nc * ns
    per_w = _N // nw

    @functools.partial(
        pl.kernel,
        mesh=plsc.VectorSubcoreMesh(core_axis_name="c", subcore_axis_name="s"),
        compiler_params=pltpu.CompilerParams(use_tc_tiling_on_sc=False),
        out_type=(jax.ShapeDtypeStruct((_N, 8), jnp.float32),
                  jax.ShapeDtypeStruct((_N, 8), jnp.float32)),
        scratch_types=[pltpu.VMEM((per_w,), jnp.int32),
                       pltpu.VMEM((per_w, 8), jnp.float32),
                       pltpu.VMEM((per_w,), jnp.int32),
                       pltpu.VMEM((per_w, 8), jnp.float32),
                       pltpu.SemaphoreType.DMA,
                       pltpu.SemaphoreType.DMA],
    )
    def gk(tok_tab_h, var_tab_h, tok_idx_h, var_idx_h, tok_out, var_out,
           ti_v, tr_v, vi_v, vr_v, s1, s2):
        wid = lax.axis_index("s") * nc + lax.axis_index("c")
        base = wid * per_w
        pltpu.sync_copy(tok_idx_h.at[pl.ds(base, per_w)], ti_v)
        pltpu.sync_copy(var_idx_h.at[pl.ds(base, per_w)], vi_v)

        def body(j, carry):
            o = j * _K
            c1 = pltpu.async_copy(tok_tab_h.at[ti_v.at[pl.ds(o, _K)]],
                                  tr_v.at[pl.ds(o, _K)], s1)
            c2 = pltpu.async_copy(var_tab_h.at[vi_v.at[pl.ds(o, _K)]],
                                  vr_v.at[pl.ds(o, _K)], s2)
            c1.wait()
            c2.wait()
            return carry

        lax.fori_loop(0, per_w // _K, body, 0)
        pltpu.sync_copy(tr_v, tok_out.at[pl.ds(base, per_w)])
        pltpu.sync_copy(vr_v, var_out.at[pl.ds(base, per_w)])

    return gk(tok_tab, var_tab, tok_idx, var_idx)


def _tc_body(tok_ref, var_ref, tid_ref, u_ref, wa_ref, wb_ref, wc_ref, wd_ref,
             hb_ref, e2_ref, bw_ref, w2_ref, b2_ref, out_ref):
    f32 = jnp.float32
    inv = np.float32(1.0 / (_S - 1))
    U = u_ref[...]
    tid = tid_ref[...]
    tchunks = [tok_ref[b] for b in range(_BB)]
    vchunks = [var_ref[b] for b in range(_BB)]
    Ts = [jnp.concatenate([tchunks[b], vchunks[b]], axis=1).T
          for b in range(_BB)]

    def rows(i):
        return jnp.concatenate([Ts[b][i:i + 1] for b in range(_BB)], axis=0)

    hi, sg, lg = rows(0), rows(1), rows(2)
    famf, grpf = rows(12), rows(13)
    valid = (hi > 0.0) & (tid != 0) & (tid != 1) & (tid != 2)
    vf = valid.astype(f32)
    lv = lg * vf
    famoh = [(famf == np.float32(f)).astype(f32) for f in range(8)]
    grpoh = [(grpf == np.float32(g)).astype(f32) for g in range(16)]
    gmask = (grpf > 0.0).astype(f32)
    vg = vf * gmask
    lvg = lv * gmask
    X = jnp.concatenate(
        [vf, lv]
        + [famoh[f] * vf for f in range(8)]
        + [famoh[f] * lv for f in range(8)]
        + [grpoh[g] * vg for g in range(16)]
        + [grpoh[g] * lvg for g in range(16)], axis=0)
    prev = jnp.dot(X, U, preferred_element_type=f32)
    pc = prev[0:_BB]
    pls = prev[_BB:2 * _BB]
    psfc = sum(prev[(2 + f) * _BB:(3 + f) * _BB] * famoh[f] for f in range(8))
    psfl = sum(prev[(10 + f) * _BB:(11 + f) * _BB] * famoh[f] for f in range(8))
    psgc = sum(prev[(18 + g) * _BB:(19 + g) * _BB] * grpoh[g] for g in range(16))
    psgl = sum(prev[(34 + g) * _BB:(35 + g) * _BB] * grpoh[g] for g in range(16))
    rden = 1.0 / jnp.maximum(pc, 1.0)
    z1 = jnp.zeros((_BB, 1), f32)
    dyn10 = [pc * inv, pls * inv, psfc * inv, psfl * inv, psfc * rden,
             psgc * inv, psgl * inv, psgc * rden,
             jnp.concatenate([z1, lg[:, :_S - 1]], axis=1),
             jnp.concatenate([z1, sg[:, :_S - 1]], axis=1)]
    dynT = jnp.concatenate(
        [jnp.concatenate([p[b:b + 1] for p in dyn10], axis=0).T
         for b in range(_BB)], axis=0)
    tok = tok_ref[...].reshape(_BB * _S, 8)
    var = var_ref[...].reshape(_BB * _S, 8)
    ioq = lax.broadcasted_iota(jnp.int32, (1, 8), 1).astype(f32)
    fam_rm = (var[:, 4:5] == ioq).astype(f32)
    wa, wb, wc, wd = wa_ref[...], wb_ref[...], wc_ref[...], wd_ref[...]
    hmu = (hb_ref[...]
           + jnp.dot(tok, wa, preferred_element_type=f32)
           + jnp.dot(var, wb, preferred_element_type=f32)
           + jnp.dot(fam_rm, wc, preferred_element_type=f32)
           + jnp.dot(dynT, wd, preferred_element_type=f32))
    mu = hmu[:, _HID:_HID + 1]
    ex2 = (e2_ref[...]
           + jnp.dot(tok * tok, wa[:, _HID:], preferred_element_type=f32)
           + jnp.dot(var * var, wb[:, _HID:], preferred_element_type=f32)
           + jnp.dot(fam_rm, wc[:, _HID:], preferred_element_type=f32)
           + jnp.dot(dynT * dynT, wd[:, _HID:], preferred_element_type=f32))
    r = lax.rsqrt(ex2 - mu * mu + 1e-5)
    h = hmu[:, :_HID] * r + bw_ref[...]
    act = 0.5 * h * (1.0 + lax.erf(h * np.float32(1.0 / np.sqrt(2.0))))
    o = jnp.dot(act, w2_ref[...], preferred_element_type=f32) + b2_ref[...]
    for b in range(_BB):
        out_ref[b] = o[b * _S:(b + 1) * _S, :]


def _tc_forward(tok3, var3, token_ids, u, wa, wb, wc, wd, hb, e2, bw, w2s, b2s):
    cb = lambda shape: pl.BlockSpec(shape, lambda i: (0,) * len(shape))
    return pl.pallas_call(
        _tc_body,
        grid=(_B // _BB,),
        in_specs=[
            pl.BlockSpec((_BB, _S, 8), lambda i: (i, 0, 0)),
            pl.BlockSpec((_BB, _S, 8), lambda i: (i, 0, 0)),
            pl.BlockSpec((_BB, _S), lambda i: (i, 0)),
            cb((_S, _S)),
            cb((8, _HID + 1)),
            cb((8, _HID + 1)),
            cb((8, _HID + 1)),
            cb((10, _HID + 1)),
            cb((_BB * _S, _HID + 1)),
            cb((_BB * _S, 1)),
            cb((1, _HID)),
            cb((_HID, _DM)),
            cb((1, _DM)),
        ],
        out_specs=pl.BlockSpec((_BB, _S, _DM), lambda i: (i, 0, 0)),
        out_shape=jax.ShapeDtypeStruct((_B, _S, _DM), jnp.float32),
    )(tok3, var3, token_ids, u, wa, wb, wc, wd, hb, e2, bw, w2s, b2s)


def kernel(token_has_int, token_signed_norm, token_log_norm, token_is_zero,
           token_is_one, token_is_pow2, var_family_onehot, var_outer_norm,
           var_inner_norm, var_has_outer, var_has_inner, ln_gamma, ln_beta,
           w1, b1, w2, b2, scale, token_ids, var_ids, var_family_id,
           var_group_id):
    f32 = jnp.float32
    zv = jnp.zeros_like(token_has_int)
    tok_tab = jnp.stack([token_has_int, token_signed_norm, token_log_norm,
                         token_is_zero, token_is_one, token_is_pow2, zv, zv],
                        axis=1)
    zn = jnp.zeros_like(var_outer_norm)
    var_tab = jnp.stack([var_outer_norm, var_inner_norm, var_has_outer,
                         var_has_inner, var_family_id.astype(f32),
                         var_group_id.astype(f32), zn, zn], axis=1)
    tok_g, var_g = _sc_gather(tok_tab, var_tab,
                              token_ids.reshape(-1), var_ids.reshape(-1))
    tok3 = tok_g.reshape(_B, _S, 8)
    var3 = var_g.reshape(_B, _S, 8)
    wts = _prep_weights(ln_gamma, ln_beta, w1, b1, w2, b2, scale)
    return _tc_forward(tok3, var3, token_ids, jnp.asarray(_UPPER), *wts)


def _prep_weights(ln_gamma, ln_beta, w1, b1, w2, b2, scale):
    f32 = jnp.float32
    i29 = np.float32(1.0 / 29.0)
    w1g = w1 * ln_gamma[:, None]
    cw = (ln_gamma @ w1).reshape(1, _HID)

    def piece(mapping):
        mp = jnp.asarray(mapping)
        use = (mp >= 0).astype(f32)[:, None]
        sel = (jnp.take(w1g, jnp.maximum(mp, 0), axis=0) - i29 * cw) * use
        return jnp.concatenate([sel, use * i29], axis=1)

    poscol = jnp.asarray(_POSCOL)
    hrow = jnp.concatenate([w1g[17:18] - i29 * cw,
                            jnp.full((1, 1), i29, f32)], axis=1)
    hb = poscol * hrow
    e2 = poscol * poscol * i29
    bw = (ln_beta @ w1 + b1).reshape(1, _HID)
    return (piece(_MAP_TOK), piece(_MAP_VAR), piece(_MAP_FAM), piece(_MAP_DYN),
            hb, e2, bw, w2 * scale, (b2 * scale).reshape(1, _DM))


# speedup vs baseline: 32.4281x; 1.0344x over previous
import functools

import numpy as np
import jax
import jax.numpy as jnp
from jax import lax
from jax.experimental import pallas as pl
from jax.experimental.pallas import tpu as pltpu
from jax.experimental.pallas import tpu_sc as plsc

_B = 1024
_S = 200
_N = _B * _S
_BB = 8
_HID = 64
_DM = 128
_K = 128

_MAP_TOK = np.array([0, 1, 2, 3, 28, 4, -1, -1], dtype=np.int32)
_MAP_VAR = np.array([13, 14, 15, 16, -1, -1, -1, -1], dtype=np.int32)
_MAP_FAM = np.arange(5, 13, dtype=np.int32)
_MAP_DYN = np.arange(18, 28, dtype=np.int32)
_UPPER = np.triu(np.ones((_S, _S), np.float32), 1)
_POSCOL = ((np.arange(_BB * _S) % _S).astype(np.float32) / (_S - 1))[:, None]


def _sc_gather(tok_tab, var_tab, tok_idx, var_idx):
    info = plsc.get_sparse_core_info()
    nc, ns = info.num_cores, info.num_subcores
    nw = nc * ns
    per_w = _N // nw

    @functools.partial(
        pl.kernel,
        mesh=plsc.VectorSubcoreMesh(core_axis_name="c", subcore_axis_name="s"),
        compiler_params=pltpu.CompilerParams(use_tc_tiling_on_sc=False),
        out_type=(jax.ShapeDtypeStruct((_N, 8), jnp.float32),
                  jax.ShapeDtypeStruct((_N, 8), jnp.float32)),
        scratch_types=[pltpu.VMEM((per_w,), jnp.int32),
                       pltpu.VMEM((per_w, 8), jnp.float32),
                       pltpu.VMEM((per_w,), jnp.int32),
                       pltpu.VMEM((per_w, 8), jnp.float32),
                       pltpu.SemaphoreType.DMA,
                       pltpu.SemaphoreType.DMA],
    )
    def gk(tok_tab_h, var_tab_h, tok_idx_h, var_idx_h, tok_out, var_out,
           ti_v, tr_v, vi_v, vr_v, s1, s2):
        wid = lax.axis_index("s") * nc + lax.axis_index("c")
        base = wid * per_w
        pltpu.sync_copy(tok_idx_h.at[pl.ds(base, per_w)], ti_v)
        pltpu.sync_copy(var_idx_h.at[pl.ds(base, per_w)], vi_v)

        grp = 5

        def body(g, carry):
            cs = []
            for k in range(grp):
                o = (g * grp + k) * _K
                cs.append(pltpu.async_copy(tok_tab_h.at[ti_v.at[pl.ds(o, _K)]],
                                           tr_v.at[pl.ds(o, _K)], s1))
                cs.append(pltpu.async_copy(var_tab_h.at[vi_v.at[pl.ds(o, _K)]],
                                           vr_v.at[pl.ds(o, _K)], s2))
            for c in cs:
                c.wait()
            return carry

        lax.fori_loop(0, per_w // (_K * grp), body, 0)
        pltpu.sync_copy(tr_v, tok_out.at[pl.ds(base, per_w)])
        pltpu.sync_copy(vr_v, var_out.at[pl.ds(base, per_w)])

    return gk(tok_tab, var_tab, tok_idx, var_idx)


def _tc_body(tok_ref, var_ref, tid_ref, u_ref, wa_ref, wb_ref, wc_ref, wd_ref,
             hb_ref, e2_ref, bw_ref, w2_ref, b2_ref, out_ref):
    f32 = jnp.float32
    inv = np.float32(1.0 / (_S - 1))
    U = u_ref[...]
    tid = tid_ref[...]
    tchunks = [tok_ref[b] for b in range(_BB)]
    vchunks = [var_ref[b] for b in range(_BB)]
    Ts = [jnp.concatenate([tchunks[b], vchunks[b]], axis=1).T
          for b in range(_BB)]

    def rows(i):
        return jnp.concatenate([Ts[b][i:i + 1] for b in range(_BB)], axis=0)

    hi, sg, lg = rows(0), rows(1), rows(2)
    famf, grpf = rows(12), rows(13)
    valid = (hi > 0.0) & (tid != 0) & (tid != 1) & (tid != 2)
    vf = valid.astype(f32)
    lv = lg * vf
    famoh = [(famf == np.float32(f)).astype(f32) for f in range(8)]
    grpoh = [(grpf == np.float32(g)).astype(f32) for g in range(16)]
    gmask = (grpf > 0.0).astype(f32)
    vg = vf * gmask
    lvg = lv * gmask
    X = jnp.concatenate(
        [vf, lv]
        + [famoh[f] * vf for f in range(8)]
        + [famoh[f] * lv for f in range(8)]
        + [grpoh[g] * vg for g in range(16)]
        + [grpoh[g] * lvg for g in range(16)], axis=0)
    prev = jnp.dot(X, U, preferred_element_type=f32)
    pc = prev[0:_BB]
    pls = prev[_BB:2 * _BB]
    psfc = sum(prev[(2 + f) * _BB:(3 + f) * _BB] * famoh[f] for f in range(8))
    psfl = sum(prev[(10 + f) * _BB:(11 + f) * _BB] * famoh[f] for f in range(8))
    psgc = sum(prev[(18 + g) * _BB:(19 + g) * _BB] * grpoh[g] for g in range(16))
    psgl = sum(prev[(34 + g) * _BB:(35 + g) * _BB] * grpoh[g] for g in range(16))
    rden = 1.0 / jnp.maximum(pc, 1.0)
    z1 = jnp.zeros((_BB, 1), f32)
    dyn10 = [pc * inv, pls * inv, psfc * inv, psfl * inv, psfc * rden,
             psgc * inv, psgl * inv, psgc * rden,
             jnp.concatenate([z1, lg[:, :_S - 1]], axis=1),
             jnp.concatenate([z1, sg[:, :_S - 1]], axis=1)]
    dynT = jnp.concatenate(
        [jnp.concatenate([p[b:b + 1] for p in dyn10], axis=0).T
         for b in range(_BB)], axis=0)
    tok = tok_ref[...].reshape(_BB * _S, 8)
    var = var_ref[...].reshape(_BB * _S, 8)
    ioq = lax.broadcasted_iota(jnp.int32, (1, 8), 1).astype(f32)
    fam_rm = (var[:, 4:5] == ioq).astype(f32)
    wa, wb, wc, wd = wa_ref[...], wb_ref[...], wc_ref[...], wd_ref[...]
    hmu = (hb_ref[...]
           + jnp.dot(tok, wa, preferred_element_type=f32)
           + jnp.dot(var, wb, preferred_element_type=f32)
           + jnp.dot(fam_rm, wc, preferred_element_type=f32)
           + jnp.dot(dynT, wd, preferred_element_type=f32))
    mu = hmu[:, _HID:_HID + 1]
    ex2 = (e2_ref[...]
           + jnp.dot(tok * tok, wa[:, _HID:], preferred_element_type=f32)
           + jnp.dot(var * var, wb[:, _HID:], preferred_element_type=f32)
           + jnp.dot(fam_rm, wc[:, _HID:], preferred_element_type=f32)
           + jnp.dot(dynT * dynT, wd[:, _HID:], preferred_element_type=f32))
    r = lax.rsqrt(ex2 - mu * mu + 1e-5)
    h = hmu[:, :_HID] * r + bw_ref[...]
    act = 0.5 * h * (1.0 + lax.erf(h * np.float32(1.0 / np.sqrt(2.0))))
    o = jnp.dot(act, w2_ref[...], preferred_element_type=f32) + b2_ref[...]
    for b in range(_BB):
        out_ref[b] = o[b * _S:(b + 1) * _S, :]


def _tc_forward(tok3, var3, token_ids, u, wa, wb, wc, wd, hb, e2, bw, w2s, b2s):
    cb = lambda shape: pl.BlockSpec(shape, lambda i: (0,) * len(shape))
    return pl.pallas_call(
        _tc_body,
        grid=(_B // _BB,),
        in_specs=[
            pl.BlockSpec((_BB, _S, 8), lambda i: (i, 0, 0)),
            pl.BlockSpec((_BB, _S, 8), lambda i: (i, 0, 0)),
            pl.BlockSpec((_BB, _S), lambda i: (i, 0)),
            cb((_S, _S)),
            cb((8, _HID + 1)),
            cb((8, _HID + 1)),
            cb((8, _HID + 1)),
            cb((10, _HID + 1)),
            cb((_BB * _S, _HID + 1)),
            cb((_BB * _S, 1)),
            cb((1, _HID)),
            cb((_HID, _DM)),
            cb((1, _DM)),
        ],
        out_specs=pl.BlockSpec((_BB, _S, _DM), lambda i: (i, 0, 0)),
        out_shape=jax.ShapeDtypeStruct((_B, _S, _DM), jnp.float32),
    )(tok3, var3, token_ids, u, wa, wb, wc, wd, hb, e2, bw, w2s, b2s)


def kernel(token_has_int, token_signed_norm, token_log_norm, token_is_zero,
           token_is_one, token_is_pow2, var_family_onehot, var_outer_norm,
           var_inner_norm, var_has_outer, var_has_inner, ln_gamma, ln_beta,
           w1, b1, w2, b2, scale, token_ids, var_ids, var_family_id,
           var_group_id):
    f32 = jnp.float32
    zv = jnp.zeros_like(token_has_int)
    tok_tab = jnp.stack([token_has_int, token_signed_norm, token_log_norm,
                         token_is_zero, token_is_one, token_is_pow2, zv, zv],
                        axis=1)
    zn = jnp.zeros_like(var_outer_norm)
    var_tab = jnp.stack([var_outer_norm, var_inner_norm, var_has_outer,
                         var_has_inner, var_family_id.astype(f32),
                         var_group_id.astype(f32), zn, zn], axis=1)
    tok_g, var_g = _sc_gather(tok_tab, var_tab,
                              token_ids.reshape(-1), var_ids.reshape(-1))
    tok3 = tok_g.reshape(_B, _S, 8)
    var3 = var_g.reshape(_B, _S, 8)
    wts = _prep_weights(ln_gamma, ln_beta, w1, b1, w2, b2, scale)
    return _tc_forward(tok3, var3, token_ids, jnp.asarray(_UPPER), *wts)


def _prep_weights(ln_gamma, ln_beta, w1, b1, w2, b2, scale):
    f32 = jnp.float32
    i29 = np.float32(1.0 / 29.0)
    w1g = w1 * ln_gamma[:, None]
    cw = (ln_gamma @ w1).reshape(1, _HID)

    def piece(mapping):
        mp = jnp.asarray(mapping)
        use = (mp >= 0).astype(f32)[:, None]
        sel = (jnp.take(w1g, jnp.maximum(mp, 0), axis=0) - i29 * cw) * use
        return jnp.concatenate([sel, use * i29], axis=1)

    poscol = jnp.asarray(_POSCOL)
    hrow = jnp.concatenate([w1g[17:18] - i29 * cw,
                            jnp.full((1, 1), i29, f32)], axis=1)
    hb = poscol * hrow
    e2 = poscol * poscol * i29
    bw = (ln_beta @ w1 + b1).reshape(1, _HID)
    return (piece(_MAP_TOK), piece(_MAP_VAR), piece(_MAP_FAM), piece(_MAP_DYN),
            hb, e2, bw, w2 * scale, (b2 * scale).reshape(1, _DM))


# speedup vs baseline: 37.1047x; 1.1442x over previous
import functools

import numpy as np
import jax
import jax.numpy as jnp
from jax import lax
from jax.experimental import pallas as pl
from jax.experimental.pallas import tpu as pltpu
from jax.experimental.pallas import tpu_sc as plsc

_B = 1024
_S = 200
_N = _B * _S
_BB = 8
_HID = 64
_DM = 128
_K = 128

_MAP_TOK = np.array([0, 1, 2, 3, 28, 4, -1, -1], dtype=np.int32)
_MAP_VAR = np.array([13, 14, 15, 16, -1, -1, -1, -1], dtype=np.int32)
_MAP_FAM = np.arange(5, 13, dtype=np.int32)
_MAP_DYN = np.arange(18, 28, dtype=np.int32)
_UPPER = np.triu(np.ones((_S, _S), np.float32), 1)
_POSCOL = ((np.arange(_BB * _S) % _S).astype(np.float32) / (_S - 1))[:, None]


def _sc_gather(tok_tab, var_tab, tok_idx, var_idx):
    info = plsc.get_sparse_core_info()
    nc, ns = info.num_cores, info.num_subcores
    nw = nc * ns
    per_w = _N // nw

    @functools.partial(
        pl.kernel,
        mesh=plsc.VectorSubcoreMesh(core_axis_name="c", subcore_axis_name="s"),
        compiler_params=pltpu.CompilerParams(use_tc_tiling_on_sc=False),
        out_type=jax.ShapeDtypeStruct((_N, 128), jnp.float32),
        scratch_types=[pltpu.VMEM((per_w,), jnp.int32),
                       pltpu.VMEM((per_w, 8), jnp.float32),
                       pltpu.VMEM((per_w,), jnp.int32),
                       pltpu.VMEM((per_w, 8), jnp.float32),
                       pltpu.SemaphoreType.DMA,
                       pltpu.SemaphoreType.DMA],
    )
    def gk(tok_tab_h, var_tab_h, tok_idx_h, var_idx_h, tv_out,
           ti_v, tr_v, vi_v, vr_v, s1, s2):
        wid = lax.axis_index("s") * nc + lax.axis_index("c")
        base = wid * per_w
        pltpu.sync_copy(tok_idx_h.at[pl.ds(base, per_w)], ti_v)
        pltpu.sync_copy(var_idx_h.at[pl.ds(base, per_w)], vi_v)

        grp = 5

        def body(g, carry):
            cs = []
            for k in range(grp):
                o = (g * grp + k) * _K
                cs.append(pltpu.async_copy(tok_tab_h.at[ti_v.at[pl.ds(o, _K)]],
                                           tr_v.at[pl.ds(o, _K)], s1))
                cs.append(pltpu.async_copy(var_tab_h.at[vi_v.at[pl.ds(o, _K)]],
                                           vr_v.at[pl.ds(o, _K)], s2))
            for c in cs:
                c.wait()
            return carry

        lax.fori_loop(0, per_w // (_K * grp), body, 0)
        pltpu.sync_copy(tr_v, tv_out.at[pl.ds(base, per_w), pl.ds(0, 8)])
        pltpu.sync_copy(vr_v, tv_out.at[pl.ds(base, per_w), pl.ds(8, 8)])

    return gk(tok_tab, var_tab, tok_idx, var_idx)


def _tc_body(tv_ref, tid_ref, u_ref, wa_ref, wb_ref, wc_ref, wd_ref,
             hb_ref, e2_ref, bw_ref, w2_ref, b2_ref, out_ref):
    f32 = jnp.float32
    inv = np.float32(1.0 / (_S - 1))
    U = u_ref[...]
    tid = tid_ref[...]
    gb = tv_ref[...]
    Ts = [gb[b * _S:(b + 1) * _S, 0:16].T for b in range(_BB)]

    def rows(i):
        return jnp.concatenate([Ts[b][i:i + 1] for b in range(_BB)], axis=0)

    hi, sg, lg = rows(0), rows(1), rows(2)
    famf, grpf = rows(12), rows(13)
    valid = (hi > 0.0) & (tid != 0) & (tid != 1) & (tid != 2)
    vf = valid.astype(f32)
    lv = lg * vf
    famoh = [(famf == np.float32(f)).astype(f32) for f in range(8)]
    grpoh = [(grpf == np.float32(g)).astype(f32) for g in range(16)]
    gmask = (grpf > 0.0).astype(f32)
    vg = vf * gmask
    lvg = lv * gmask
    X = jnp.concatenate(
        [vf, lv]
        + [famoh[f] * vf for f in range(8)]
        + [famoh[f] * lv for f in range(8)]
        + [grpoh[g] * vg for g in range(16)]
        + [grpoh[g] * lvg for g in range(16)], axis=0)
    prev = jnp.dot(X, U, preferred_element_type=f32)
    pc = prev[0:_BB]
    pls = prev[_BB:2 * _BB]
    psfc = sum(prev[(2 + f) * _BB:(3 + f) * _BB] * famoh[f] for f in range(8))
    psfl = sum(prev[(10 + f) * _BB:(11 + f) * _BB] * famoh[f] for f in range(8))
    psgc = sum(prev[(18 + g) * _BB:(19 + g) * _BB] * grpoh[g] for g in range(16))
    psgl = sum(prev[(34 + g) * _BB:(35 + g) * _BB] * grpoh[g] for g in range(16))
    rden = 1.0 / jnp.maximum(pc, 1.0)
    z1 = jnp.zeros((_BB, 1), f32)
    dyn10 = [pc * inv, pls * inv, psfc * inv, psfl * inv, psfc * rden,
             psgc * inv, psgl * inv, psgc * rden,
             jnp.concatenate([z1, lg[:, :_S - 1]], axis=1),
             jnp.concatenate([z1, sg[:, :_S - 1]], axis=1)]
    dynT = jnp.concatenate(
        [jnp.concatenate([p[b:b + 1] for p in dyn10], axis=0).T
         for b in range(_BB)], axis=0)
    tok = gb[:, 0:8]
    var = gb[:, 8:16]
    ioq = lax.broadcasted_iota(jnp.int32, (1, 8), 1).astype(f32)
    fam_rm = (var[:, 4:5] == ioq).astype(f32)
    wa, wb, wc, wd = wa_ref[...], wb_ref[...], wc_ref[...], wd_ref[...]
    hmu = (hb_ref[...]
           + jnp.dot(tok, wa, preferred_element_type=f32)
           + jnp.dot(var, wb, preferred_element_type=f32)
           + jnp.dot(fam_rm, wc, preferred_element_type=f32)
           + jnp.dot(dynT, wd, preferred_element_type=f32))
    mu = hmu[:, _HID:_HID + 1]
    ex2 = (e2_ref[...]
           + jnp.dot(tok * tok, wa[:, _HID:], preferred_element_type=f32)
           + jnp.dot(var * var, wb[:, _HID:], preferred_element_type=f32)
           + jnp.dot(fam_rm, wc[:, _HID:], preferred_element_type=f32)
           + jnp.dot(dynT * dynT, wd[:, _HID:], preferred_element_type=f32))
    r = lax.rsqrt(ex2 - mu * mu + 1e-5)
    h = hmu[:, :_HID] * r + bw_ref[...]
    act = 0.5 * h * (1.0 + lax.erf(h * np.float32(1.0 / np.sqrt(2.0))))
    o = jnp.dot(act, w2_ref[...], preferred_element_type=f32) + b2_ref[...]
    for b in range(_BB):
        out_ref[b] = o[b * _S:(b + 1) * _S, :]


def _tc_forward(tv, token_ids, u, wa, wb, wc, wd, hb, e2, bw, w2s, b2s):
    cb = lambda shape: pl.BlockSpec(shape, lambda i: (0,) * len(shape))
    return pl.pallas_call(
        _tc_body,
        grid=(_B // _BB,),
        in_specs=[
            pl.BlockSpec((_BB * _S, 128), lambda i: (i, 0)),
            pl.BlockSpec((_BB, _S), lambda i: (i, 0)),
            cb((_S, _S)),
            cb((8, _HID + 1)),
            cb((8, _HID + 1)),
            cb((8, _HID + 1)),
            cb((10, _HID + 1)),
            cb((_BB * _S, _HID + 1)),
            cb((_BB * _S, 1)),
            cb((1, _HID)),
            cb((_HID, _DM)),
            cb((1, _DM)),
        ],
        out_specs=pl.BlockSpec((_BB, _S, _DM), lambda i: (i, 0, 0)),
        out_shape=jax.ShapeDtypeStruct((_B, _S, _DM), jnp.float32),
    )(tv, token_ids, u, wa, wb, wc, wd, hb, e2, bw, w2s, b2s)


def kernel(token_has_int, token_signed_norm, token_log_norm, token_is_zero,
           token_is_one, token_is_pow2, var_family_onehot, var_outer_norm,
           var_inner_norm, var_has_outer, var_has_inner, ln_gamma, ln_beta,
           w1, b1, w2, b2, scale, token_ids, var_ids, var_family_id,
           var_group_id):
    f32 = jnp.float32
    zv = jnp.zeros_like(token_has_int)
    tok_tab = jnp.stack([token_has_int, token_signed_norm, token_log_norm,
                         token_is_zero, token_is_one, token_is_pow2, zv, zv],
                        axis=1)
    zn = jnp.zeros_like(var_outer_norm)
    var_tab = jnp.stack([var_outer_norm, var_inner_norm, var_has_outer,
                         var_has_inner, var_family_id.astype(f32),
                         var_group_id.astype(f32), zn, zn], axis=1)
    tv = _sc_gather(tok_tab, var_tab,
                    token_ids.reshape(-1), var_ids.reshape(-1))
    wts = _prep_weights(ln_gamma, ln_beta, w1, b1, w2, b2, scale)
    return _tc_forward(tv, token_ids, jnp.asarray(_UPPER), *wts)


def _prep_weights(ln_gamma, ln_beta, w1, b1, w2, b2, scale):
    f32 = jnp.float32
    i29 = np.float32(1.0 / 29.0)
    w1g = w1 * ln_gamma[:, None]
    cw = (ln_gamma @ w1).reshape(1, _HID)

    def piece(mapping):
        mp = jnp.asarray(mapping)
        use = (mp >= 0).astype(f32)[:, None]
        sel = (jnp.take(w1g, jnp.maximum(mp, 0), axis=0) - i29 * cw) * use
        return jnp.concatenate([sel, use * i29], axis=1)

    poscol = jnp.asarray(_POSCOL)
    hrow = jnp.concatenate([w1g[17:18] - i29 * cw,
                            jnp.full((1, 1), i29, f32)], axis=1)
    hb = poscol * hrow
    e2 = poscol * poscol * i29
    bw = (ln_beta @ w1 + b1).reshape(1, _HID)
    return (piece(_MAP_TOK), piece(_MAP_VAR), piece(_MAP_FAM), piece(_MAP_DYN),
            hb, e2, bw, w2 * scale, (b2 * scale).reshape(1, _DM))


# speedup vs baseline: 42.5276x; 1.1461x over previous
import functools

import numpy as np
import jax
import jax.numpy as jnp
from jax import lax
from jax.experimental import pallas as pl
from jax.experimental.pallas import tpu as pltpu
from jax.experimental.pallas import tpu_sc as plsc

_B = 1024
_S = 200
_N = _B * _S
_BB = 16
_HID = 64
_DM = 128
_K = 128

_MAP_TOK = np.array([0, 1, 2, 3, 28, 4, -1, -1], dtype=np.int32)
_MAP_VAR = np.array([13, 14, 15, 16, -1, -1, -1, -1], dtype=np.int32)
_MAP_FAM = np.arange(5, 13, dtype=np.int32)
_MAP_DYN = np.arange(18, 28, dtype=np.int32)
_UPPER = np.triu(np.ones((_S, _S), np.float32), 1)
_POSCOL = ((np.arange(_BB * _S) % _S).astype(np.float32) / (_S - 1))[:, None]


def _sc_gather(tok_tab, var_tab, tok_idx, var_idx):
    info = plsc.get_sparse_core_info()
    nc, ns = info.num_cores, info.num_subcores
    nw = nc * ns
    per_w = _N // nw

    @functools.partial(
        pl.kernel,
        mesh=plsc.VectorSubcoreMesh(core_axis_name="c", subcore_axis_name="s"),
        compiler_params=pltpu.CompilerParams(use_tc_tiling_on_sc=False),
        out_type=jax.ShapeDtypeStruct((_N, 128), jnp.float32),
        scratch_types=[pltpu.VMEM((per_w,), jnp.int32),
                       pltpu.VMEM((per_w, 8), jnp.float32),
                       pltpu.VMEM((per_w,), jnp.int32),
                       pltpu.VMEM((per_w, 8), jnp.float32),
                       pltpu.SemaphoreType.DMA,
                       pltpu.SemaphoreType.DMA],
    )
    def gk(tok_tab_h, var_tab_h, tok_idx_h, var_idx_h, tv_out,
           ti_v, tr_v, vi_v, vr_v, s1, s2):
        wid = lax.axis_index("s") * nc + lax.axis_index("c")
        base = wid * per_w
        pltpu.sync_copy(tok_idx_h.at[pl.ds(base, per_w)], ti_v)
        pltpu.sync_copy(var_idx_h.at[pl.ds(base, per_w)], vi_v)

        grp = 5

        def body(g, carry):
            cs = []
            for k in range(grp):
                o = (g * grp + k) * _K
                cs.append(pltpu.async_copy(tok_tab_h.at[ti_v.at[pl.ds(o, _K)]],
                                           tr_v.at[pl.ds(o, _K)], s1))
                cs.append(pltpu.async_copy(var_tab_h.at[vi_v.at[pl.ds(o, _K)]],
                                           vr_v.at[pl.ds(o, _K)], s2))
            for c in cs:
                c.wait()
            return carry

        lax.fori_loop(0, per_w // (_K * grp), body, 0)
        pltpu.sync_copy(tr_v, tv_out.at[pl.ds(base, per_w), pl.ds(0, 8)])
        pltpu.sync_copy(vr_v, tv_out.at[pl.ds(base, per_w), pl.ds(8, 8)])

    return gk(tok_tab, var_tab, tok_idx, var_idx)


def _tc_body(tv_ref, tid_ref, u_ref, wa_ref, wb_ref, wc_ref, wd_ref,
             hb_ref, e2_ref, bw_ref, w2_ref, b2_ref, out_ref):
    f32 = jnp.float32
    inv = np.float32(1.0 / (_S - 1))
    U = u_ref[...]
    tid = tid_ref[...]
    gb = tv_ref[...]
    Ts = [gb[b * _S:(b + 1) * _S, 0:16].T for b in range(_BB)]

    def rows(i):
        return jnp.concatenate([Ts[b][i:i + 1] for b in range(_BB)], axis=0)

    hi, sg, lg = rows(0), rows(1), rows(2)
    famf, grpf = rows(12), rows(13)
    valid = (hi > 0.0) & (tid != 0) & (tid != 1) & (tid != 2)
    vf = valid.astype(f32)
    lv = lg * vf
    famoh = [(famf == np.float32(f)).astype(f32) for f in range(8)]
    grpoh = [(grpf == np.float32(g)).astype(f32) for g in range(16)]
    gmask = (grpf > 0.0).astype(f32)
    vg = vf * gmask
    lvg = lv * gmask
    X = jnp.concatenate(
        [vf, lv]
        + [famoh[f] * vf for f in range(8)]
        + [famoh[f] * lv for f in range(8)]
        + [grpoh[g] * vg for g in range(16)]
        + [grpoh[g] * lvg for g in range(16)], axis=0)
    prev = jnp.dot(X, U, preferred_element_type=f32)
    pc = prev[0:_BB]
    pls = prev[_BB:2 * _BB]
    psfc = sum(prev[(2 + f) * _BB:(3 + f) * _BB] * famoh[f] for f in range(8))
    psfl = sum(prev[(10 + f) * _BB:(11 + f) * _BB] * famoh[f] for f in range(8))
    psgc = sum(prev[(18 + g) * _BB:(19 + g) * _BB] * grpoh[g] for g in range(16))
    psgl = sum(prev[(34 + g) * _BB:(35 + g) * _BB] * grpoh[g] for g in range(16))
    rden = 1.0 / jnp.maximum(pc, 1.0)
    z1 = jnp.zeros((_BB, 1), f32)
    dyn10 = [pc * inv, pls * inv, psfc * inv, psfl * inv, psfc * rden,
             psgc * inv, psgl * inv, psgc * rden,
             jnp.concatenate([z1, lg[:, :_S - 1]], axis=1),
             jnp.concatenate([z1, sg[:, :_S - 1]], axis=1)]
    dynT = jnp.concatenate(
        [jnp.concatenate([p[b:b + 1] for p in dyn10], axis=0).T
         for b in range(_BB)], axis=0)
    tok = gb[:, 0:8]
    var = gb[:, 8:16]
    ioq = lax.broadcasted_iota(jnp.int32, (1, 8), 1).astype(f32)
    fam_rm = (var[:, 4:5] == ioq).astype(f32)
    wa, wb, wc, wd = wa_ref[...], wb_ref[...], wc_ref[...], wd_ref[...]
    hmu = (hb_ref[...]
           + jnp.dot(tok, wa, preferred_element_type=f32)
           + jnp.dot(var, wb, preferred_element_type=f32)
           + jnp.dot(fam_rm, wc, preferred_element_type=f32)
           + jnp.dot(dynT, wd, preferred_element_type=f32))
    mu = hmu[:, _HID:_HID + 1]
    ex2 = (e2_ref[...]
           + jnp.dot(tok * tok, wa[:, _HID:], preferred_element_type=f32)
           + jnp.dot(var * var, wb[:, _HID:], preferred_element_type=f32)
           + jnp.dot(fam_rm, wc[:, _HID:], preferred_element_type=f32)
           + jnp.dot(dynT * dynT, wd[:, _HID:], preferred_element_type=f32))
    r = lax.rsqrt(ex2 - mu * mu + 1e-5)
    h = hmu[:, :_HID] * r + bw_ref[...]
    act = 0.5 * h * (1.0 + lax.erf(h * np.float32(1.0 / np.sqrt(2.0))))
    o = jnp.dot(act, w2_ref[...], preferred_element_type=f32) + b2_ref[...]
    for b in range(_BB):
        out_ref[b] = o[b * _S:(b + 1) * _S, :]


def _tc_forward(tv, token_ids, u, wa, wb, wc, wd, hb, e2, bw, w2s, b2s):
    cb = lambda shape: pl.BlockSpec(shape, lambda i: (0,) * len(shape))
    return pl.pallas_call(
        _tc_body,
        grid=(_B // _BB,),
        in_specs=[
            pl.BlockSpec((_BB * _S, 128), lambda i: (i, 0)),
            pl.BlockSpec((_BB, _S), lambda i: (i, 0)),
            cb((_S, _S)),
            cb((8, _HID + 1)),
            cb((8, _HID + 1)),
            cb((8, _HID + 1)),
            cb((10, _HID + 1)),
            cb((_BB * _S, _HID + 1)),
            cb((_BB * _S, 1)),
            cb((1, _HID)),
            cb((_HID, _DM)),
            cb((1, _DM)),
        ],
        out_specs=pl.BlockSpec((_BB, _S, _DM), lambda i: (i, 0, 0)),
        out_shape=jax.ShapeDtypeStruct((_B, _S, _DM), jnp.float32),
    )(tv, token_ids, u, wa, wb, wc, wd, hb, e2, bw, w2s, b2s)


def kernel(token_has_int, token_signed_norm, token_log_norm, token_is_zero,
           token_is_one, token_is_pow2, var_family_onehot, var_outer_norm,
           var_inner_norm, var_has_outer, var_has_inner, ln_gamma, ln_beta,
           w1, b1, w2, b2, scale, token_ids, var_ids, var_family_id,
           var_group_id):
    f32 = jnp.float32
    zv = jnp.zeros_like(token_has_int)
    tok_tab = jnp.stack([token_has_int, token_signed_norm, token_log_norm,
                         token_is_zero, token_is_one, token_is_pow2, zv, zv],
                        axis=1)
    zn = jnp.zeros_like(var_outer_norm)
    var_tab = jnp.stack([var_outer_norm, var_inner_norm, var_has_outer,
                         var_has_inner, var_family_id.astype(f32),
                         var_group_id.astype(f32), zn, zn], axis=1)
    tv = _sc_gather(tok_tab, var_tab,
                    token_ids.reshape(-1), var_ids.reshape(-1))
    wts = _prep_weights(ln_gamma, ln_beta, w1, b1, w2, b2, scale)
    return _tc_forward(tv, token_ids, jnp.asarray(_UPPER), *wts)


def _prep_weights(ln_gamma, ln_beta, w1, b1, w2, b2, scale):
    f32 = jnp.float32
    i29 = np.float32(1.0 / 29.0)
    w1g = w1 * ln_gamma[:, None]
    cw = (ln_gamma @ w1).reshape(1, _HID)

    def piece(mapping):
        mp = jnp.asarray(mapping)
        use = (mp >= 0).astype(f32)[:, None]
        sel = (jnp.take(w1g, jnp.maximum(mp, 0), axis=0) - i29 * cw) * use
        return jnp.concatenate([sel, use * i29], axis=1)

    poscol = jnp.asarray(_POSCOL)
    hrow = jnp.concatenate([w1g[17:18] - i29 * cw,
                            jnp.full((1, 1), i29, f32)], axis=1)
    hb = poscol * hrow
    e2 = poscol * poscol * i29
    bw = (ln_beta @ w1 + b1).reshape(1, _HID)
    return (piece(_MAP_TOK), piece(_MAP_VAR), piece(_MAP_FAM), piece(_MAP_DYN),
            hb, e2, bw, w2 * scale, (b2 * scale).reshape(1, _DM))


# speedup vs baseline: 42.9009x; 1.0088x over previous
import functools

import numpy as np
import jax
import jax.numpy as jnp
from jax import lax
from jax.experimental import pallas as pl
from jax.experimental.pallas import tpu as pltpu
from jax.experimental.pallas import tpu_sc as plsc

_B = 1024
_S = 200
_N = _B * _S
_BB = 32
_HID = 64
_DM = 128
_K = 128

_MAP_TOK = np.array([0, 1, 2, 3, 28, 4, -1, -1], dtype=np.int32)
_MAP_VAR = np.array([13, 14, 15, 16, -1, -1, -1, -1], dtype=np.int32)
_MAP_FAM = np.arange(5, 13, dtype=np.int32)
_MAP_DYN = np.arange(18, 28, dtype=np.int32)
_UPPER = np.triu(np.ones((_S, _S), np.float32), 1)
_POSCOL = ((np.arange(_BB * _S) % _S).astype(np.float32) / (_S - 1))[:, None]


def _sc_gather(tok_tab, var_tab, tok_idx, var_idx):
    info = plsc.get_sparse_core_info()
    nc, ns = info.num_cores, info.num_subcores
    nw = nc * ns
    per_w = _N // nw

    @functools.partial(
        pl.kernel,
        mesh=plsc.VectorSubcoreMesh(core_axis_name="c", subcore_axis_name="s"),
        compiler_params=pltpu.CompilerParams(use_tc_tiling_on_sc=False),
        out_type=jax.ShapeDtypeStruct((_N, 128), jnp.float32),
        scratch_types=[pltpu.VMEM((per_w,), jnp.int32),
                       pltpu.VMEM((per_w, 8), jnp.float32),
                       pltpu.VMEM((per_w,), jnp.int32),
                       pltpu.VMEM((per_w, 8), jnp.float32),
                       pltpu.SemaphoreType.DMA,
                       pltpu.SemaphoreType.DMA],
    )
    def gk(tok_tab_h, var_tab_h, tok_idx_h, var_idx_h, tv_out,
           ti_v, tr_v, vi_v, vr_v, s1, s2):
        wid = lax.axis_index("s") * nc + lax.axis_index("c")
        base = wid * per_w
        pltpu.sync_copy(tok_idx_h.at[pl.ds(base, per_w)], ti_v)
        pltpu.sync_copy(var_idx_h.at[pl.ds(base, per_w)], vi_v)

        grp = 5

        def body(g, carry):
            cs = []
            for k in range(grp):
                o = (g * grp + k) * _K
                cs.append(pltpu.async_copy(tok_tab_h.at[ti_v.at[pl.ds(o, _K)]],
                                           tr_v.at[pl.ds(o, _K)], s1))
                cs.append(pltpu.async_copy(var_tab_h.at[vi_v.at[pl.ds(o, _K)]],
                                           vr_v.at[pl.ds(o, _K)], s2))
            for c in cs:
                c.wait()
            return carry

        lax.fori_loop(0, per_w // (_K * grp), body, 0)
        pltpu.sync_copy(tr_v, tv_out.at[pl.ds(base, per_w), pl.ds(0, 8)])
        pltpu.sync_copy(vr_v, tv_out.at[pl.ds(base, per_w), pl.ds(8, 8)])

    return gk(tok_tab, var_tab, tok_idx, var_idx)


def _tc_body(tv_ref, tid_ref, u_ref, wa_ref, wb_ref, wc_ref, wd_ref,
             hb_ref, e2_ref, bw_ref, w2_ref, b2_ref, out_ref):
    f32 = jnp.float32
    inv = np.float32(1.0 / (_S - 1))
    U = u_ref[...]
    tid = tid_ref[...]
    gb = tv_ref[...]
    Ts = [gb[b * _S:(b + 1) * _S, 0:16].T for b in range(_BB)]

    def rows(i):
        return jnp.concatenate([Ts[b][i:i + 1] for b in range(_BB)], axis=0)

    hi, sg, lg = rows(0), rows(1), rows(2)
    famf, grpf = rows(12), rows(13)
    valid = (hi > 0.0) & (tid != 0) & (tid != 1) & (tid != 2)
    vf = valid.astype(f32)
    lv = lg * vf
    famoh = [(famf == np.float32(f)).astype(f32) for f in range(8)]
    grpoh = [(grpf == np.float32(g)).astype(f32) for g in range(16)]
    gmask = (grpf > 0.0).astype(f32)
    vg = vf * gmask
    lvg = lv * gmask
    X = jnp.concatenate(
        [vf, lv]
        + [famoh[f] * vf for f in range(8)]
        + [famoh[f] * lv for f in range(8)]
        + [grpoh[g] * vg for g in range(16)]
        + [grpoh[g] * lvg for g in range(16)], axis=0)
    prev = jnp.dot(X, U, preferred_element_type=f32)
    pc = prev[0:_BB]
    pls = prev[_BB:2 * _BB]
    psfc = sum(prev[(2 + f) * _BB:(3 + f) * _BB] * famoh[f] for f in range(8))
    psfl = sum(prev[(10 + f) * _BB:(11 + f) * _BB] * famoh[f] for f in range(8))
    psgc = sum(prev[(18 + g) * _BB:(19 + g) * _BB] * grpoh[g] for g in range(16))
    psgl = sum(prev[(34 + g) * _BB:(35 + g) * _BB] * grpoh[g] for g in range(16))
    rden = 1.0 / jnp.maximum(pc, 1.0)
    z1 = jnp.zeros((_BB, 1), f32)
    dyn10 = [pc * inv, pls * inv, psfc * inv, psfl * inv, psfc * rden,
             psgc * inv, psgl * inv, psgc * rden,
             jnp.concatenate([z1, lg[:, :_S - 1]], axis=1),
             jnp.concatenate([z1, sg[:, :_S - 1]], axis=1)]
    dynT = jnp.concatenate(
        [jnp.concatenate([p[b:b + 1] for p in dyn10], axis=0).T
         for b in range(_BB)], axis=0)
    tok = gb[:, 0:8]
    var = gb[:, 8:16]
    ioq = lax.broadcasted_iota(jnp.int32, (1, 8), 1).astype(f32)
    fam_rm = (var[:, 4:5] == ioq).astype(f32)
    wa, wb, wc, wd = wa_ref[...], wb_ref[...], wc_ref[...], wd_ref[...]
    hmu = (hb_ref[...]
           + jnp.dot(tok, wa, preferred_element_type=f32)
           + jnp.dot(var, wb, preferred_element_type=f32)
           + jnp.dot(fam_rm, wc, preferred_element_type=f32)
           + jnp.dot(dynT, wd, preferred_element_type=f32))
    mu = hmu[:, _HID:_HID + 1]
    ex2 = (e2_ref[...]
           + jnp.dot(tok * tok, wa[:, _HID:], preferred_element_type=f32)
           + jnp.dot(var * var, wb[:, _HID:], preferred_element_type=f32)
           + jnp.dot(fam_rm, wc[:, _HID:], preferred_element_type=f32)
           + jnp.dot(dynT * dynT, wd[:, _HID:], preferred_element_type=f32))
    r = lax.rsqrt(ex2 - mu * mu + 1e-5)
    h = hmu[:, :_HID] * r + bw_ref[...]
    act = 0.5 * h * (1.0 + lax.erf(h * np.float32(1.0 / np.sqrt(2.0))))
    o = jnp.dot(act, w2_ref[...], preferred_element_type=f32) + b2_ref[...]
    for b in range(_BB):
        out_ref[b] = o[b * _S:(b + 1) * _S, :]


def _tc_forward(tv, token_ids, u, wa, wb, wc, wd, hb, e2, bw, w2s, b2s):
    cb = lambda shape: pl.BlockSpec(shape, lambda i: (0,) * len(shape))
    return pl.pallas_call(
        _tc_body,
        grid=(_B // _BB,),
        in_specs=[
            pl.BlockSpec((_BB * _S, 128), lambda i: (i, 0)),
            pl.BlockSpec((_BB, _S), lambda i: (i, 0)),
            cb((_S, _S)),
            cb((8, _HID + 1)),
            cb((8, _HID + 1)),
            cb((8, _HID + 1)),
            cb((10, _HID + 1)),
            cb((_BB * _S, _HID + 1)),
            cb((_BB * _S, 1)),
            cb((1, _HID)),
            cb((_HID, _DM)),
            cb((1, _DM)),
        ],
        out_specs=pl.BlockSpec((_BB, _S, _DM), lambda i: (i, 0, 0)),
        out_shape=jax.ShapeDtypeStruct((_B, _S, _DM), jnp.float32),
    )(tv, token_ids, u, wa, wb, wc, wd, hb, e2, bw, w2s, b2s)


def kernel(token_has_int, token_signed_norm, token_log_norm, token_is_zero,
           token_is_one, token_is_pow2, var_family_onehot, var_outer_norm,
           var_inner_norm, var_has_outer, var_has_inner, ln_gamma, ln_beta,
           w1, b1, w2, b2, scale, token_ids, var_ids, var_family_id,
           var_group_id):
    f32 = jnp.float32
    zv = jnp.zeros_like(token_has_int)
    tok_tab = jnp.stack([token_has_int, token_signed_norm, token_log_norm,
                         token_is_zero, token_is_one, token_is_pow2, zv, zv],
                        axis=1)
    zn = jnp.zeros_like(var_outer_norm)
    var_tab = jnp.stack([var_outer_norm, var_inner_norm, var_has_outer,
                         var_has_inner, var_family_id.astype(f32),
                         var_group_id.astype(f32), zn, zn], axis=1)
    tv = _sc_gather(tok_tab, var_tab,
                    token_ids.reshape(-1), var_ids.reshape(-1))
    wts = _prep_weights(ln_gamma, ln_beta, w1, b1, w2, b2, scale)
    return _tc_forward(tv, token_ids, jnp.asarray(_UPPER), *wts)


def _prep_weights(ln_gamma, ln_beta, w1, b1, w2, b2, scale):
    f32 = jnp.float32
    i29 = np.float32(1.0 / 29.0)
    w1g = w1 * ln_gamma[:, None]
    cw = (ln_gamma @ w1).reshape(1, _HID)

    def piece(mapping):
        mp = jnp.asarray(mapping)
        use = (mp >= 0).astype(f32)[:, None]
        sel = (jnp.take(w1g, jnp.maximum(mp, 0), axis=0) - i29 * cw) * use
        return jnp.concatenate([sel, use * i29], axis=1)

    poscol = jnp.asarray(_POSCOL)
    hrow = jnp.concatenate([w1g[17:18] - i29 * cw,
                            jnp.full((1, 1), i29, f32)], axis=1)
    hb = poscol * hrow
    e2 = poscol * poscol * i29
    bw = (ln_beta @ w1 + b1).reshape(1, _HID)
    return (piece(_MAP_TOK), piece(_MAP_VAR), piece(_MAP_FAM), piece(_MAP_DYN),
            hb, e2, bw, w2 * scale, (b2 * scale).reshape(1, _DM))


# speedup vs baseline: 43.6528x; 1.0175x over previous
import functools

import numpy as np
import jax
import jax.numpy as jnp
from jax import lax
from jax.experimental import pallas as pl
from jax.experimental.pallas import tpu as pltpu
from jax.experimental.pallas import tpu_sc as plsc

_B = 1024
_S = 200
_N = _B * _S
_BB = 32
_HID = 64
_DM = 128
_K = 128
_GRP = 5

_MAP_TOK = np.array([0, 1, 2, 3, 28, 4, -1, -1], dtype=np.int32)
_MAP_VAR = np.array([13, 14, 15, 16, -1, -1, -1, -1], dtype=np.int32)
_MAP_FAM = np.arange(5, 13, dtype=np.int32)
_MAP_DYN = np.arange(18, 28, dtype=np.int32)
_UPPER = np.triu(np.ones((_S, _S), np.float32), 1)
_POSCOL = ((np.arange(_BB * _S) % _S).astype(np.float32) / (_S - 1))[:, None]


def _sc_gather(tok_tab, var_tab, tok_idx, var_idx):
    info = plsc.get_sparse_core_info()
    nc, ns = info.num_cores, info.num_subcores
    nw = nc * ns
    per_w = _N // nw

    @functools.partial(
        pl.kernel,
        mesh=plsc.VectorSubcoreMesh(core_axis_name="c", subcore_axis_name="s"),
        compiler_params=pltpu.CompilerParams(use_tc_tiling_on_sc=False),
        out_type=jax.ShapeDtypeStruct((_N, 128), jnp.float32),
        scratch_types=[pltpu.VMEM((per_w,), jnp.int32),
                       pltpu.VMEM((_GRP * _K, 16), jnp.float32),
                       pltpu.VMEM((per_w,), jnp.int32),
                       pltpu.VMEM((_GRP * _K, 16), jnp.float32),
                       pltpu.SemaphoreType.DMA,
                       pltpu.SemaphoreType.DMA],
    )
    def gk(tok_tab_h, var_tab_h, tok_idx_h, var_idx_h, tv_out,
           ti_v, tr_v, vi_v, vr_v, s1, s2):
        wid = lax.axis_index("s") * nc + lax.axis_index("c")
        base = wid * per_w
        pltpu.sync_copy(tok_idx_h.at[pl.ds(base, per_w)], ti_v)
        pltpu.sync_copy(var_idx_h.at[pl.ds(base, per_w)], vi_v)

        grp = _GRP

        def body(g, carry):
            cs = []
            for k in range(grp):
                o = (g * grp + k) * _K
                cs.append(pltpu.async_copy(tok_tab_h.at[ti_v.at[pl.ds(o, _K)]],
                                           tr_v.at[pl.ds(k * _K, _K)], s1))
                cs.append(pltpu.async_copy(var_tab_h.at[vi_v.at[pl.ds(o, _K)]],
                                           vr_v.at[pl.ds(k * _K, _K)], s2))
            for c in cs:
                c.wait()
            o0 = base + g * grp * _K
            pltpu.sync_copy(tr_v, tv_out.at[pl.ds(o0, grp * _K), pl.ds(0, 16)])
            pltpu.sync_copy(vr_v, tv_out.at[pl.ds(o0, grp * _K), pl.ds(16, 16)])
            return carry

        lax.fori_loop(0, per_w // (_K * grp), body, 0)

    return gk(tok_tab, var_tab, tok_idx, var_idx)


def _tc_body(tv_ref, tid_ref, u_ref, wa_ref, wb_ref, wc_ref, wd_ref,
             hb_ref, e2_ref, bw_ref, w2_ref, b2_ref, out_ref):
    f32 = jnp.float32
    inv = np.float32(1.0 / (_S - 1))
    U = u_ref[...]
    tid = tid_ref[...]
    gb = tv_ref[...]
    Ts = [gb[b * _S:(b + 1) * _S, 0:24].T for b in range(_BB)]

    def rows(i):
        return jnp.concatenate([Ts[b][i:i + 1] for b in range(_BB)], axis=0)

    hi, sg, lg = rows(0), rows(1), rows(2)
    famf, grpf = rows(20), rows(21)
    valid = (hi > 0.0) & (tid != 0) & (tid != 1) & (tid != 2)
    vf = valid.astype(f32)
    lv = lg * vf
    famoh = [(famf == np.float32(f)).astype(f32) for f in range(8)]
    grpoh = [(grpf == np.float32(g)).astype(f32) for g in range(16)]
    gmask = (grpf > 0.0).astype(f32)
    vg = vf * gmask
    lvg = lv * gmask
    X = jnp.concatenate(
        [vf, lv]
        + [famoh[f] * vf for f in range(8)]
        + [famoh[f] * lv for f in range(8)]
        + [grpoh[g] * vg for g in range(16)]
        + [grpoh[g] * lvg for g in range(16)], axis=0)
    prev = jnp.dot(X, U, preferred_element_type=f32)
    pc = prev[0:_BB]
    pls = prev[_BB:2 * _BB]
    psfc = sum(prev[(2 + f) * _BB:(3 + f) * _BB] * famoh[f] for f in range(8))
    psfl = sum(prev[(10 + f) * _BB:(11 + f) * _BB] * famoh[f] for f in range(8))
    psgc = sum(prev[(18 + g) * _BB:(19 + g) * _BB] * grpoh[g] for g in range(16))
    psgl = sum(prev[(34 + g) * _BB:(35 + g) * _BB] * grpoh[g] for g in range(16))
    rden = 1.0 / jnp.maximum(pc, 1.0)
    z1 = jnp.zeros((_BB, 1), f32)
    dyn10 = [pc * inv, pls * inv, psfc * inv, psfl * inv, psfc * rden,
             psgc * inv, psgl * inv, psgc * rden,
             jnp.concatenate([z1, lg[:, :_S - 1]], axis=1),
             jnp.concatenate([z1, sg[:, :_S - 1]], axis=1)]
    dynT = jnp.concatenate(
        [jnp.concatenate([p[b:b + 1] for p in dyn10], axis=0).T
         for b in range(_BB)], axis=0)
    tok = gb[:, 0:8]
    var = gb[:, 16:24]
    ioq = lax.broadcasted_iota(jnp.int32, (1, 8), 1).astype(f32)
    fam_rm = (var[:, 4:5] == ioq).astype(f32)
    wa, wb, wc, wd = wa_ref[...], wb_ref[...], wc_ref[...], wd_ref[...]
    hmu = (hb_ref[...]
           + jnp.dot(tok, wa, preferred_element_type=f32)
           + jnp.dot(var, wb, preferred_element_type=f32)
           + jnp.dot(fam_rm, wc, preferred_element_type=f32)
           + jnp.dot(dynT, wd, preferred_element_type=f32))
    mu = hmu[:, _HID:_HID + 1]
    ex2 = (e2_ref[...]
           + jnp.dot(tok * tok, wa[:, _HID:], preferred_element_type=f32)
           + jnp.dot(var * var, wb[:, _HID:], preferred_element_type=f32)
           + jnp.dot(fam_rm, wc[:, _HID:], preferred_element_type=f32)
           + jnp.dot(dynT * dynT, wd[:, _HID:], preferred_element_type=f32))
    r = lax.rsqrt(ex2 - mu * mu + 1e-5)
    h = hmu[:, :_HID] * r + bw_ref[...]
    act = 0.5 * h * (1.0 + lax.erf(h * np.float32(1.0 / np.sqrt(2.0))))
    o = jnp.dot(act, w2_ref[...], preferred_element_type=f32) + b2_ref[...]
    for b in range(_BB):
        out_ref[b] = o[b * _S:(b + 1) * _S, :]


def _tc_forward(tv, token_ids, u, wa, wb, wc, wd, hb, e2, bw, w2s, b2s):
    cb = lambda shape: pl.BlockSpec(shape, lambda i: (0,) * len(shape))
    return pl.pallas_call(
        _tc_body,
        grid=(_B // _BB,),
        in_specs=[
            pl.BlockSpec((_BB * _S, 128), lambda i: (i, 0)),
            pl.BlockSpec((_BB, _S), lambda i: (i, 0)),
            cb((_S, _S)),
            cb((8, _HID + 1)),
            cb((8, _HID + 1)),
            cb((8, _HID + 1)),
            cb((10, _HID + 1)),
            cb((_BB * _S, _HID + 1)),
            cb((_BB * _S, 1)),
            cb((1, _HID)),
            cb((_HID, _DM)),
            cb((1, _DM)),
        ],
        out_specs=pl.BlockSpec((_BB, _S, _DM), lambda i: (i, 0, 0)),
        out_shape=jax.ShapeDtypeStruct((_B, _S, _DM), jnp.float32),
    )(tv, token_ids, u, wa, wb, wc, wd, hb, e2, bw, w2s, b2s)


def kernel(token_has_int, token_signed_norm, token_log_norm, token_is_zero,
           token_is_one, token_is_pow2, var_family_onehot, var_outer_norm,
           var_inner_norm, var_has_outer, var_has_inner, ln_gamma, ln_beta,
           w1, b1, w2, b2, scale, token_ids, var_ids, var_family_id,
           var_group_id):
    f32 = jnp.float32
    zv = jnp.zeros_like(token_has_int)
    tok_tab = jnp.stack([token_has_int, token_signed_norm, token_log_norm,
                         token_is_zero, token_is_one, token_is_pow2,
                         zv, zv, zv, zv, zv, zv, zv, zv, zv, zv], axis=1)
    zn = jnp.zeros_like(var_outer_norm)
    var_tab = jnp.stack([var_outer_norm, var_inner_norm, var_has_outer,
                         var_has_inner, var_family_id.astype(f32),
                         var_group_id.astype(f32),
                         zn, zn, zn, zn, zn, zn, zn, zn, zn, zn], axis=1)
    tv = _sc_gather(tok_tab, var_tab,
                    token_ids.reshape(-1), var_ids.reshape(-1))
    wts = _prep_weights(ln_gamma, ln_beta, w1, b1, w2, b2, scale)
    return _tc_forward(tv, token_ids, jnp.asarray(_UPPER), *wts)


def _prep_weights(ln_gamma, ln_beta, w1, b1, w2, b2, scale):
    f32 = jnp.float32
    i29 = np.float32(1.0 / 29.0)
    w1g = w1 * ln_gamma[:, None]
    cw = (ln_gamma @ w1).reshape(1, _HID)

    def piece(mapping):
        mp = jnp.asarray(mapping)
        use = (mp >= 0).astype(f32)[:, None]
        sel = (jnp.take(w1g, jnp.maximum(mp, 0), axis=0) - i29 * cw) * use
        return jnp.concatenate([sel, use * i29], axis=1)

    poscol = jnp.asarray(_POSCOL)
    hrow = jnp.concatenate([w1g[17:18] - i29 * cw,
                            jnp.full((1, 1), i29, f32)], axis=1)
    hb = poscol * hrow
    e2 = poscol * poscol * i29
    bw = (ln_beta @ w1 + b1).reshape(1, _HID)
    return (piece(_MAP_TOK), piece(_MAP_VAR), piece(_MAP_FAM), piece(_MAP_DYN),
            hb, e2, bw, w2 * scale, (b2 * scale).reshape(1, _DM))


# speedup vs baseline: 61.2257x; 1.4026x over previous
import functools

import numpy as np
import jax
import jax.numpy as jnp
from jax import lax
from jax.experimental import pallas as pl
from jax.experimental.pallas import tpu as pltpu
from jax.experimental.pallas import tpu_sc as plsc

_B = 1024
_S = 200
_N = _B * _S
_BB = 32
_HID = 64
_DM = 128
_K = 128
_GRP = 5

_MAP_TOK = np.array([0, 1, 2, 3, 28, 4, -1, -1], dtype=np.int32)
_MAP_VAR = np.array([13, 14, 15, 16, -1, -1, -1, -1], dtype=np.int32)
_MAP_FAM = np.arange(5, 13, dtype=np.int32)
_MAP_DYN = np.arange(18, 28, dtype=np.int32)
_UPPER = np.triu(np.ones((_S, _S), np.float32), 1)
_POSCOL = ((np.arange(_BB * _S) % _S).astype(np.float32) / (_S - 1))[:, None]


def _sc_gather(tok_tab, var_tab, tok_idx, var_idx):
    info = plsc.get_sparse_core_info()
    nc, ns = info.num_cores, info.num_subcores
    nw = nc * ns
    per_w = _N // nw

    @functools.partial(
        pl.kernel,
        mesh=plsc.VectorSubcoreMesh(core_axis_name="c", subcore_axis_name="s"),
        compiler_params=pltpu.CompilerParams(use_tc_tiling_on_sc=False),
        out_type=jax.ShapeDtypeStruct((_N, 128), jnp.float32),
        scratch_types=[pltpu.VMEM((per_w,), jnp.int32),
                       pltpu.VMEM((_GRP * _K, 16), jnp.float32),
                       pltpu.VMEM((per_w,), jnp.int32),
                       pltpu.VMEM((_GRP * _K, 16), jnp.float32),
                       pltpu.SemaphoreType.DMA,
                       pltpu.SemaphoreType.DMA],
    )
    def gk(tok_tab_h, var_tab_h, tok_idx_h, var_idx_h, tv_out,
           ti_v, tr_v, vi_v, vr_v, s1, s2):
        wid = lax.axis_index("s") * nc + lax.axis_index("c")
        base = wid * per_w
        pltpu.sync_copy(tok_idx_h.at[pl.ds(base, per_w)], ti_v)
        pltpu.sync_copy(var_idx_h.at[pl.ds(base, per_w)], vi_v)

        grp = _GRP

        def body(g, carry):
            cs = []
            for k in range(grp):
                o = (g * grp + k) * _K
                cs.append(pltpu.async_copy(tok_tab_h.at[ti_v.at[pl.ds(o, _K)]],
                                           tr_v.at[pl.ds(k * _K, _K)], s1))
                cs.append(pltpu.async_copy(var_tab_h.at[vi_v.at[pl.ds(o, _K)]],
                                           vr_v.at[pl.ds(k * _K, _K)], s2))
            for c in cs:
                c.wait()
            o0 = base + g * grp * _K
            pltpu.sync_copy(tr_v, tv_out.at[pl.ds(o0, grp * _K), pl.ds(0, 16)])
            pltpu.sync_copy(vr_v, tv_out.at[pl.ds(o0, grp * _K), pl.ds(16, 16)])
            return carry

        lax.fori_loop(0, per_w // (_K * grp), body, 0)

    return gk(tok_tab, var_tab, tok_idx, var_idx)


def _tc_body(tv_ref, tid_ref, u_ref, w32_ref, wdh_ref, wdq_ref,
             hb_ref, bw_ref, w2_ref, b2_ref, out_ref):
    f32 = jnp.float32
    inv = np.float32(1.0 / (_S - 1))
    U = u_ref[...]
    tid = tid_ref[...]
    gb = tv_ref[...]
    Ts = [gb[b * _S:(b + 1) * _S, 0:24].T for b in range(_BB)]

    def rows(i):
        return jnp.concatenate([Ts[b][i:i + 1] for b in range(_BB)], axis=0)

    hi, sg, lg = rows(0), rows(1), rows(2)
    famf, grpf = rows(20), rows(21)
    valid = (hi > 0.0) & (tid != 0) & (tid != 1) & (tid != 2)
    vf = valid.astype(f32)
    lv = lg * vf
    famoh = [(famf == np.float32(f)).astype(f32) for f in range(8)]
    grpoh = [(grpf == np.float32(g)).astype(f32) for g in range(16)]
    gmask = (grpf > 0.0).astype(f32)
    vg = vf * gmask
    lvg = lv * gmask
    X = jnp.concatenate(
        [vf, lv]
        + [famoh[f] * vf for f in range(8)]
        + [famoh[f] * lv for f in range(8)]
        + [grpoh[g] * vg for g in range(16)]
        + [grpoh[g] * lvg for g in range(16)], axis=0)
    prev = jnp.dot(X, U, preferred_element_type=f32)
    pc = prev[0:_BB]
    pls = prev[_BB:2 * _BB]
    psfc = sum(prev[(2 + f) * _BB:(3 + f) * _BB] * famoh[f] for f in range(8))
    psfl = sum(prev[(10 + f) * _BB:(11 + f) * _BB] * famoh[f] for f in range(8))
    psgc = sum(prev[(18 + g) * _BB:(19 + g) * _BB] * grpoh[g] for g in range(16))
    psgl = sum(prev[(34 + g) * _BB:(35 + g) * _BB] * grpoh[g] for g in range(16))
    rden = 1.0 / jnp.maximum(pc, 1.0)
    z1 = jnp.zeros((_BB, 1), f32)
    dyn10 = [pc * inv, pls * inv, psfc * inv, psfl * inv, psfc * rden,
             psgc * inv, psgl * inv, psgc * rden,
             jnp.concatenate([z1, lg[:, :_S - 1]], axis=1),
             jnp.concatenate([z1, sg[:, :_S - 1]], axis=1)]
    dynT = jnp.concatenate(
        [jnp.concatenate([p[b:b + 1] for p in dyn10], axis=0).T
         for b in range(_BB)], axis=0)
    g32 = gb[:, 0:32]
    hmu = (hb_ref[...]
           + jnp.dot(g32, w32_ref[...], preferred_element_type=f32)
           + jnp.dot(dynT, wdh_ref[...], preferred_element_type=f32)
           + jnp.dot(dynT * dynT, wdq_ref[...], preferred_element_type=f32))
    mu = hmu[:, _HID:_HID + 1]
    ex2 = hmu[:, _HID + 1:_HID + 2]
    r = lax.rsqrt(ex2 - mu * mu + 1e-5)
    h = hmu[:, :_HID] * r + bw_ref[...]
    act = 0.5 * h * (1.0 + lax.erf(h * np.float32(1.0 / np.sqrt(2.0))))
    o = jnp.dot(act, w2_ref[...], preferred_element_type=f32) + b2_ref[...]
    for b in range(_BB):
        out_ref[b] = o[b * _S:(b + 1) * _S, :]


def _tc_forward(tv, token_ids, u, w32, wdh, wdq, hb, bw, w2s, b2s):
    cb = lambda shape: pl.BlockSpec(shape, lambda i: (0,) * len(shape))
    return pl.pallas_call(
        _tc_body,
        grid=(_B // _BB,),
        in_specs=[
            pl.BlockSpec((_BB * _S, 128), lambda i: (i, 0)),
            pl.BlockSpec((_BB, _S), lambda i: (i, 0)),
            cb((_S, _S)),
            cb((32, _HID + 2)),
            cb((10, _HID + 2)),
            cb((10, _HID + 2)),
            cb((_BB * _S, _HID + 2)),
            cb((1, _HID)),
            cb((_HID, _DM)),
            cb((1, _DM)),
        ],
        out_specs=pl.BlockSpec((_BB, _S, _DM), lambda i: (i, 0, 0)),
        out_shape=jax.ShapeDtypeStruct((_B, _S, _DM), jnp.float32),
    )(tv, token_ids, u, w32, wdh, wdq, hb, bw, w2s, b2s)


def kernel(token_has_int, token_signed_norm, token_log_norm, token_is_zero,
           token_is_one, token_is_pow2, var_family_onehot, var_outer_norm,
           var_inner_norm, var_has_outer, var_has_inner, ln_gamma, ln_beta,
           w1, b1, w2, b2, scale, token_ids, var_ids, var_family_id,
           var_group_id):
    tok_tab, var_tab = _build_tables(
        token_has_int, token_signed_norm, token_log_norm, token_is_zero,
        token_is_one, token_is_pow2, var_family_onehot, var_outer_norm,
        var_inner_norm, var_has_outer, var_has_inner, var_family_id,
        var_group_id)
    tv = _sc_gather(tok_tab, var_tab,
                    token_ids.reshape(-1), var_ids.reshape(-1))
    wts = _prep_weights(ln_gamma, ln_beta, w1, b1, w2, b2, scale)
    return _tc_forward(tv, token_ids, jnp.asarray(_UPPER), *wts)


def _build_tables(token_has_int, token_signed_norm, token_log_norm,
                  token_is_zero, token_is_one, token_is_pow2,
                  var_family_onehot, var_outer_norm, var_inner_norm,
                  var_has_outer, var_has_inner, var_family_id, var_group_id):
    f32 = jnp.float32
    i29 = np.float32(1.0 / 29.0)
    zv = jnp.zeros_like(token_has_int)
    ind = token_has_int + token_is_zero + token_is_one + token_is_pow2
    s1t = (ind + token_signed_norm + token_log_norm) * i29
    s2t = (ind + token_signed_norm ** 2 + token_log_norm ** 2) * i29
    tok_tab = jnp.stack([token_has_int, token_signed_norm, token_log_norm,
                         token_is_zero, token_is_one, token_is_pow2,
                         zv, zv, s1t, s2t, zv, zv, zv, zv, zv, zv], axis=1)
    vind = var_has_outer + var_has_inner + 1.0
    s1v = (vind + var_outer_norm + var_inner_norm) * i29
    s2v = (vind + var_outer_norm ** 2 + var_inner_norm ** 2) * i29
    var_tab = jnp.concatenate(
        [jnp.stack([var_outer_norm, var_inner_norm, var_has_outer,
                    var_has_inner, var_family_id.astype(f32),
                    var_group_id.astype(f32), s1v, s2v], axis=1),
         var_family_onehot], axis=1)
    return tok_tab, var_tab


_LANE32 = np.array([0, 1, 2, 3, 4, 5, 16, 17, 18, 19,
                    24, 25, 26, 27, 28, 29, 30, 31], dtype=np.int32)
_ORIG32 = np.array([0, 1, 2, 3, 28, 4, 13, 14, 15, 16,
                    5, 6, 7, 8, 9, 10, 11, 12], dtype=np.int32)


def _prep_weights(ln_gamma, ln_beta, w1, b1, w2, b2, scale):
    f32 = jnp.float32
    i29 = np.float32(1.0 / 29.0)
    w1g = w1 * ln_gamma[:, None]
    cw = (ln_gamma @ w1).reshape(1, _HID)
    wp = w1g - i29 * cw
    wpe = jnp.concatenate([wp, jnp.zeros((29, 2), f32)], axis=1)
    w32 = jnp.zeros((32, _HID + 2), f32)
    w32 = w32.at[jnp.asarray(_LANE32)].set(wpe[jnp.asarray(_ORIG32)])
    w32 = w32.at[8, _HID].set(1.0).at[9, _HID + 1].set(1.0)
    w32 = w32.at[22, _HID].set(1.0).at[23, _HID + 1].set(1.0)
    wdh = jnp.concatenate([wp[18:28], jnp.full((10, 1), i29, f32),
                           jnp.zeros((10, 1), f32)], axis=1)
    wdq = jnp.concatenate([jnp.zeros((10, _HID + 1), f32),
                           jnp.full((10, 1), i29, f32)], axis=1)
    poscol = jnp.asarray(_POSCOL)
    hrow = jnp.concatenate([wp[17:18], jnp.full((1, 1), i29, f32),
                            jnp.zeros((1, 1), f32)], axis=1)
    hb = poscol * hrow
    hb = hb.at[:, _HID + 1].set(poscol[:, 0] * poscol[:, 0] * i29)
    bw = (ln_beta @ w1 + b1).reshape(1, _HID)
    return (w32, wdh, wdq, hb, bw, w2 * scale, (b2 * scale).reshape(1, _DM))
